# baseline (speedup 1.0000x reference)
import functools

import jax
import jax.numpy as jnp
from jax import lax
from jax.experimental import pallas as pl
from jax.experimental.pallas import tpu as pltpu
from jax.experimental.pallas import tpu_sc as plsc

D = 32
BATCH = 16384
NROWS = 1000000
NC = 2
NS = 16
NW = NC * NS
BPW = BATCH // NW
CH = 1024
NFULL = NROWS // CH
LAST_LO = NFULL * CH
TAIL0 = 999936
TAIL_LO = 999872
HCAP = 2048
TCAP = 256
STAGE = (BATCH + 1) * D
TRASH = BATCH * D
TAIL_W = 17


def _sweep_body(uid_hbm, iid_hbm, jid_hbm, ufT_hbm, itfT_hbm,
                uftT_hbm, ittT_hbm,
                us_hbm, is_hbm, js_hbm,
                idb_v, hid_u, hb_u, hid_i, hb_i, hid_j, hb_j,
                tid_u, tb_u, tid_i, tb_i, tid_j, tb_j,
                chid_v, chb_v, slab_v, rows_v, ridx_v,
                sem, sem2):
    wid = lax.axis_index("s") * NC + lax.axis_index("c")
    lane = lax.iota(jnp.int32, 16)

    def prescan(list_hbm, hid, hb, tidl, tbl):
        def outer(ci, offs):
            pltpu.sync_copy(list_hbm.at[pl.ds(ci * 2048, 2048)], idb_v)

            def inner(g, offs):
                off, toff = offs
                ids = idb_v[pl.ds(g * 16, 16)]
                b = ci * 2048 + g * 16 + lane
                own = lax.shift_right_logical(ids, 10)
                m = (lax.bitwise_and(own, 31) == wid) & (ids < TAIL0)
                cnt = jnp.sum(m.astype(jnp.int32))

                @pl.when(cnt > 0)
                def _():
                    plsc.store_compressed(hid.at[pl.ds(off, 16)], ids, mask=m)
                    plsc.store_compressed(hb.at[pl.ds(off, 16)], b, mask=m)

                mt = (ids >= TAIL0) & (wid == TAIL_W)
                tcnt = jnp.sum(mt.astype(jnp.int32))

                @pl.when(tcnt > 0)
                def _():
                    plsc.store_compressed(tidl.at[pl.ds(toff, 16)], ids, mask=mt)
                    plsc.store_compressed(tbl.at[pl.ds(toff, 16)], b, mask=mt)

                return (jnp.minimum(off + cnt, HCAP - 16),
                        jnp.minimum(toff + tcnt, TCAP - 16))

            return lax.fori_loop(0, 128, inner, offs)

        return lax.fori_loop(0, 8, outer, (0, 0))

    n_u, nt_u = prescan(uid_hbm, hid_u, hb_u, tid_u, tb_u)
    n_i, nt_i = prescan(iid_hbm, hid_i, hb_i, tid_i, tb_i)
    n_j, nt_j = prescan(jid_hbm, hid_j, hb_j, tid_j, tb_j)

    neg1 = jnp.full((16,), -1, jnp.int32)

    def reset_ridx():
        def rr(q, _):
            for h in range(8):
                ridx_v[q, pl.ds(h * 16, 16)] = (
                    jnp.full((16,), TRASH, jnp.int32) + lane + h * 16)
            return _
        lax.fori_loop(0, 32, rr, 0)

    def flush(stage_hbm):
        for q in range(32):
            pltpu.make_async_copy(
                rows_v.at[pl.ds(q * 128, 128)],
                stage_hbm.at[plsc.Indices(ridx_v.at[q])],
                sem2).start()
        pltpu.make_async_copy(
            rows_v, stage_hbm.at[pl.ds(0, 128 * D)], sem2).wait()

    def extract(stage_hbm, nchunk, lo):
        def wave(w, _):
            reset_ridx()
            base = w * 128
            nrows = jnp.minimum(nchunk - base, 128)

            def grp(g, _):
                valid = (g * 16 + lane) < nrows
                ids = chid_v[pl.ds(base + g * 16, 16)]
                bs = chb_v[pl.ds(base + g * 16, 16)]
                col = jnp.where(valid, ids - lo, 0)
                slot = g * 16 + lane
                for c in range(D):
                    cc = jnp.full((16,), c, jnp.int32)
                    vals = plsc.load_gather(slab_v, [cc, col], mask=valid)
                    fpos = slot * D + c
                    plsc.store_scatter(rows_v, [fpos], vals, mask=valid)
                    hpos = bs * D + c
                    plsc.store_scatter(
                        ridx_v,
                        [lax.shift_right_logical(fpos, 7),
                         lax.bitwise_and(fpos, 127)],
                        hpos, mask=valid)
                return _

            lax.fori_loop(0, lax.div(nrows + 15, 16), grp, 0)
            flush(stage_hbm)
            return _

        lax.fori_loop(0, lax.div(nchunk + 127, 128), wave, 0)

    def scanlist(hid, hb, n, t):
        def blk(bk, coff):
            valid = (bk * 16 + lane) < n
            ids = hid[pl.ds(bk * 16, 16)]
            bs = hb[pl.ds(bk * 16, 16)]
            m = valid & (lax.shift_right_logical(ids, 10) == t)
            cnt = jnp.sum(m.astype(jnp.int32))

            @pl.when(cnt > 0)
            def _():
                plsc.store_compressed(chid_v.at[pl.ds(coff, 16)], ids, mask=m)
                plsc.store_compressed(chb_v.at[pl.ds(coff, 16)], bs, mask=m)

            return coff + cnt

        return lax.fori_loop(0, lax.div(n + 15, 16), blk, 0)

    def copy_slab(table_hbm, lo, width):
        pltpu.sync_copy(table_hbm.at[:, pl.ds(lo, width)],
                        slab_v.at[:, pl.ds(0, width)])

    def chunk_loop(s, carry):
        t = wid + 32 * s

        @pl.when(t < NFULL)
        def _():
            lo = pl.multiple_of(t * CH, CH)
            copy_slab(ufT_hbm, lo, CH)
            extract(us_hbm, scanlist(hid_u, hb_u, n_u, t), lo)
            copy_slab(itfT_hbm, lo, CH)
            extract(is_hbm, scanlist(hid_i, hb_i, n_i, t), lo)
            extract(js_hbm, scanlist(hid_j, hb_j, n_j, t), lo)

        return carry

    lax.fori_loop(0, (NFULL + NW - 1) // NW, chunk_loop, 0)

    @pl.when(wid == NFULL % NW)
    def _():
        copy_slab(ufT_hbm, LAST_LO, TAIL0 - LAST_LO)
        extract(us_hbm, scanlist(hid_u, hb_u, n_u, NFULL), LAST_LO)
        copy_slab(itfT_hbm, LAST_LO, TAIL0 - LAST_LO)
        extract(is_hbm, scanlist(hid_i, hb_i, n_i, NFULL), LAST_LO)
        extract(js_hbm, scanlist(hid_j, hb_j, n_j, NFULL), LAST_LO)

    @pl.when(wid == TAIL_W)
    def _():
        def dotail(tidl, tbl, nt, tslab_hbm, stage_hbm):
            pltpu.sync_copy(tslab_hbm, slab_v.at[:, pl.ds(0, 128)])

            def mk(g, _):
                chid_v[pl.ds(g * 16, 16)] = tidl[pl.ds(g * 16, 16)]
                chb_v[pl.ds(g * 16, 16)] = tbl[pl.ds(g * 16, 16)]
                return _

            lax.fori_loop(0, TCAP // 16, mk, 0)
            extract(stage_hbm, nt, TAIL_LO)

        dotail(tid_u, tb_u, nt_u, uftT_hbm, us_hbm)
        dotail(tid_i, tb_i, nt_i, ittT_hbm, is_hbm)
        dotail(tid_j, tb_j, nt_j, ittT_hbm, js_hbm)

    del neg1


def _compute_body(us_hbm, is_hbm, js_hbm, out_i_hbm, out_j_hbm,
                  su_v, si_v, sj_v, oi_v, oj_v):
    wid = lax.axis_index("s") * NC + lax.axis_index("c")
    base = wid * BPW
    pltpu.sync_copy(us_hbm.at[pl.ds(base * D, BPW * D)], su_v)
    pltpu.sync_copy(is_hbm.at[pl.ds(base * D, BPW * D)], si_v)
    pltpu.sync_copy(js_hbm.at[pl.ds(base * D, BPW * D)], sj_v)

    lane = lax.iota(jnp.int32, 16)
    last = lane == 15

    def body(r, _):
        u0 = su_v[pl.ds(r * D, 16)]
        u1 = su_v[pl.ds(r * D + 16, 16)]
        i0 = si_v[pl.ds(r * D, 16)]
        i1 = si_v[pl.ds(r * D + 16, 16)]
        j0 = sj_v[pl.ds(r * D, 16)]
        j1 = sj_v[pl.ds(r * D + 16, 16)]
        ci = plsc.cumsum(u0 * i0 + u1 * i1)
        cj = plsc.cumsum(u0 * j0 + u1 * j1)
        idx = jnp.full((16,), r, jnp.int32)
        plsc.store_scatter(oi_v, [idx], ci, mask=last)
        plsc.store_scatter(oj_v, [idx], cj, mask=last)
        return _

    lax.fori_loop(0, BPW, body, 0)

    pltpu.sync_copy(oi_v, out_i_hbm.at[pl.ds(base, BPW)])
    pltpu.sync_copy(oj_v, out_j_hbm.at[pl.ds(base, BPW)])


@jax.jit
def _bpr_sc(user_ids, item_ids_i, item_ids_j, user_factors, item_factors):
    mesh = plsc.VectorSubcoreMesh(core_axis_name="c", subcore_axis_name="s")
    sweep = pl.kernel(
        _sweep_body,
        out_type=(jax.ShapeDtypeStruct((STAGE,), jnp.float32),
                  jax.ShapeDtypeStruct((STAGE,), jnp.float32),
                  jax.ShapeDtypeStruct((STAGE,), jnp.float32)),
        mesh=mesh,
        compiler_params=pltpu.CompilerParams(needs_layout_passes=False,
                                             use_tc_tiling_on_sc=True),
        scratch_types=[
            pltpu.VMEM((2048,), jnp.int32),
            pltpu.VMEM((HCAP,), jnp.int32), pltpu.VMEM((HCAP,), jnp.int32),
            pltpu.VMEM((HCAP,), jnp.int32), pltpu.VMEM((HCAP,), jnp.int32),
            pltpu.VMEM((HCAP,), jnp.int32), pltpu.VMEM((HCAP,), jnp.int32),
            pltpu.VMEM((TCAP,), jnp.int32), pltpu.VMEM((TCAP,), jnp.int32),
            pltpu.VMEM((TCAP,), jnp.int32), pltpu.VMEM((TCAP,), jnp.int32),
            pltpu.VMEM((TCAP,), jnp.int32), pltpu.VMEM((TCAP,), jnp.int32),
            pltpu.VMEM((HCAP,), jnp.int32), pltpu.VMEM((HCAP,), jnp.int32),
            pltpu.VMEM((D, CH), jnp.float32),
            pltpu.VMEM((128 * D,), jnp.float32),
            pltpu.VMEM((32, 128), jnp.int32),
            pltpu.SemaphoreType.DMA,
            pltpu.SemaphoreType.DMA,
        ],
    )
    uftT = user_factors.T[:, TAIL_LO:]
    ittT = item_factors.T[:, TAIL_LO:]
    us, is_, js = sweep(user_ids, item_ids_i, item_ids_j,
                        user_factors.T, item_factors.T, uftT, ittT)

    comp = pl.kernel(
        _compute_body,
        out_type=(jax.ShapeDtypeStruct((BATCH,), jnp.float32),
                  jax.ShapeDtypeStruct((BATCH,), jnp.float32)),
        mesh=mesh,
        compiler_params=pltpu.CompilerParams(needs_layout_passes=False,
                                             use_tc_tiling_on_sc=False),
        scratch_types=[
            pltpu.VMEM((BPW * D,), jnp.float32),
            pltpu.VMEM((BPW * D,), jnp.float32),
            pltpu.VMEM((BPW * D,), jnp.float32),
            pltpu.VMEM((BPW,), jnp.float32),
            pltpu.VMEM((BPW,), jnp.float32),
        ],
    )
    return comp(us, is_, js)


def kernel(user_ids, item_ids_i, item_ids_j, user_factors, item_factors):
    return _bpr_sc(user_ids, item_ids_i, item_ids_j,
                   user_factors, item_factors)


# speedup vs baseline: 1.0003x; 1.0003x over previous
import functools

import jax
import jax.numpy as jnp
from jax import lax
from jax.experimental import pallas as pl
from jax.experimental.pallas import tpu as pltpu
from jax.experimental.pallas import tpu_sc as plsc

D = 32
BATCH = 16384
NROWS = 1000000
NC = 2
NS = 16
NW = NC * NS
BPW = BATCH // NW
CH = 1024
NFULL = NROWS // CH
LAST_LO = NFULL * CH
TAIL0 = 999936
TAIL_LO = 999872
HCAP = 2048
TCAP = 256
STAGE = (BATCH + 1) * D
TRASH = BATCH * D
TAIL_W = 17


def _sweep_body(uid_hbm, iid_hbm, jid_hbm, ufT_hbm, itfT_hbm,
                uftT_hbm, ittT_hbm,
                us_hbm, is_hbm, js_hbm,
                idb_v, hid_u, hb_u, hid_i, hb_i, hid_j, hb_j,
                tid_u, tb_u, tid_i, tb_i, tid_j, tb_j,
                chid_v, chb_v, slab_v, rows_v, ridx_v,
                sem, sem2):
    wid = lax.axis_index("s") * NC + lax.axis_index("c")
    lane = lax.iota(jnp.int32, 16)

    def prescan(list_hbm, hid, hb, tidl, tbl):
        def outer(ci, offs):
            pltpu.sync_copy(list_hbm.at[pl.ds(ci * 2048, 2048)], idb_v)

            def inner(g, offs):
                off, toff = offs
                ids = idb_v[pl.ds(g * 16, 16)]
                b = ci * 2048 + g * 16 + lane
                own = lax.shift_right_logical(ids, 10)
                m = (lax.bitwise_and(own, 31) == wid) & (ids < TAIL0)
                cnt = jnp.sum(m.astype(jnp.int32))

                @pl.when(cnt > 0)
                def _():
                    plsc.store_compressed(hid.at[pl.ds(off, 16)], ids, mask=m)
                    plsc.store_compressed(hb.at[pl.ds(off, 16)], b, mask=m)

                mt = (ids >= TAIL0) & (wid == TAIL_W)
                tcnt = jnp.sum(mt.astype(jnp.int32))

                @pl.when(tcnt > 0)
                def _():
                    plsc.store_compressed(tidl.at[pl.ds(toff, 16)], ids,
                                          mask=mt)
                    plsc.store_compressed(tbl.at[pl.ds(toff, 16)], b, mask=mt)

                return (jnp.minimum(off + cnt, HCAP - 16),
                        jnp.minimum(toff + tcnt, TCAP - 16))

            return lax.fori_loop(0, 128, inner, offs)

        return lax.fori_loop(0, 8, outer, (0, 0))

    n_u, nt_u = prescan(uid_hbm, hid_u, hb_u, tid_u, tb_u)
    n_i, nt_i = prescan(iid_hbm, hid_i, hb_i, tid_i, tb_i)
    n_j, nt_j = prescan(jid_hbm, hid_j, hb_j, tid_j, tb_j)

    def reset_ridx():
        def rr(q, _):
            for h in range(8):
                ridx_v[q, pl.ds(h * 16, 16)] = (
                    jnp.full((16,), TRASH, jnp.int32) + lane + h * 16)
            return _
        lax.fori_loop(0, 32, rr, 0)

    def flush(stage_hbm):
        for q in range(32):
            pltpu.make_async_copy(
                rows_v.at[pl.ds(q * 128, 128)],
                stage_hbm.at[plsc.Indices(ridx_v.at[q])],
                sem2).start()
        pltpu.make_async_copy(
            rows_v, stage_hbm.at[pl.ds(0, 128 * D)], sem2).wait()

    def extract(stage_hbm, nchunk, lo):
        def wave(w, _):
            reset_ridx()
            base = w * 128
            nrows = jnp.minimum(nchunk - base, 128)

            def grp(g, _):
                valid = (g * 16 + lane) < nrows
                ids = chid_v[pl.ds(base + g * 16, 16)]
                bs = chb_v[pl.ds(base + g * 16, 16)]
                col = jnp.where(valid, ids - lo, 0)
                slot = g * 16 + lane
                for c in range(D):
                    cc = jnp.full((16,), c, jnp.int32)
                    vals = plsc.load_gather(slab_v, [cc, col], mask=valid)
                    fpos = slot * D + c
                    plsc.store_scatter(rows_v, [fpos], vals, mask=valid)
                    hpos = bs * D + c
                    plsc.store_scatter(
                        ridx_v,
                        [lax.shift_right_logical(fpos, 7),
                         lax.bitwise_and(fpos, 127)],
                        hpos, mask=valid)
                return _

            lax.fori_loop(0, lax.div(nrows + 15, 16), grp, 0)
            flush(stage_hbm)
            return _

        lax.fori_loop(0, lax.div(nchunk + 127, 128), wave, 0)

    def scanlist(hid, hb, n, t):
        def blk(bk, coff):
            valid = (bk * 16 + lane) < n
            ids = hid[pl.ds(bk * 16, 16)]
            bs = hb[pl.ds(bk * 16, 16)]
            m = valid & (lax.shift_right_logical(ids, 10) == t)
            cnt = jnp.sum(m.astype(jnp.int32))

            @pl.when(cnt > 0)
            def _():
                plsc.store_compressed(chid_v.at[pl.ds(coff, 16)], ids, mask=m)
                plsc.store_compressed(chb_v.at[pl.ds(coff, 16)], bs, mask=m)

            return coff + cnt

        return lax.fori_loop(0, lax.div(n + 15, 16), blk, 0)

    def copy_slab(table_hbm, lo, width):
        cps = []
        for c8 in range(D // 8):
            cps.append(pltpu.make_async_copy(
                table_hbm.at[pl.ds(c8 * 8, 8), pl.ds(lo, width)],
                slab_v.at[pl.ds(c8 * 8, 8), pl.ds(0, width)], sem))
        for cp in cps:
            cp.start()
        for cp in cps:
            cp.wait()

    def chunk_loop(s, carry):
        t = wid + 32 * s

        @pl.when(t < NFULL)
        def _():
            lo = pl.multiple_of(t * CH, CH)
            copy_slab(ufT_hbm, lo, CH)
            extract(us_hbm, scanlist(hid_u, hb_u, n_u, t), lo)
            copy_slab(itfT_hbm, lo, CH)
            extract(is_hbm, scanlist(hid_i, hb_i, n_i, t), lo)
            extract(js_hbm, scanlist(hid_j, hb_j, n_j, t), lo)

        return carry

    lax.fori_loop(0, (NFULL + NW - 1) // NW, chunk_loop, 0)

    @pl.when(wid == NFULL % NW)
    def _():
        copy_slab(ufT_hbm, LAST_LO, TAIL0 - LAST_LO)
        extract(us_hbm, scanlist(hid_u, hb_u, n_u, NFULL), LAST_LO)
        copy_slab(itfT_hbm, LAST_LO, TAIL0 - LAST_LO)
        extract(is_hbm, scanlist(hid_i, hb_i, n_i, NFULL), LAST_LO)
        extract(js_hbm, scanlist(hid_j, hb_j, n_j, NFULL), LAST_LO)

    @pl.when(wid == TAIL_W)
    def _():
        def dotail(tidl, tbl, nt, tslab_hbm, stage_hbm):
            pltpu.sync_copy(tslab_hbm, slab_v.at[:, pl.ds(0, 128)])

            def mk(g, _):
                chid_v[pl.ds(g * 16, 16)] = tidl[pl.ds(g * 16, 16)]
                chb_v[pl.ds(g * 16, 16)] = tbl[pl.ds(g * 16, 16)]
                return _

            lax.fori_loop(0, TCAP // 16, mk, 0)
            extract(stage_hbm, nt, TAIL_LO)

        dotail(tid_u, tb_u, nt_u, uftT_hbm, us_hbm)
        dotail(tid_i, tb_i, nt_i, ittT_hbm, is_hbm)
        dotail(tid_j, tb_j, nt_j, ittT_hbm, js_hbm)


def _compute_body(us_hbm, is_hbm, js_hbm, out_i_hbm, out_j_hbm,
                  su_v, si_v, sj_v, oi_v, oj_v):
    wid = lax.axis_index("s") * NC + lax.axis_index("c")
    base = wid * BPW
    pltpu.sync_copy(us_hbm.at[pl.ds(base * D, BPW * D)], su_v)
    pltpu.sync_copy(is_hbm.at[pl.ds(base * D, BPW * D)], si_v)
    pltpu.sync_copy(js_hbm.at[pl.ds(base * D, BPW * D)], sj_v)

    lane = lax.iota(jnp.int32, 16)
    last = lane == 15

    def body(r, _):
        u0 = su_v[pl.ds(r * D, 16)]
        u1 = su_v[pl.ds(r * D + 16, 16)]
        i0 = si_v[pl.ds(r * D, 16)]
        i1 = si_v[pl.ds(r * D + 16, 16)]
        j0 = sj_v[pl.ds(r * D, 16)]
        j1 = sj_v[pl.ds(r * D + 16, 16)]
        ci = plsc.cumsum(u0 * i0 + u1 * i1)
        cj = plsc.cumsum(u0 * j0 + u1 * j1)
        idx = jnp.full((16,), r, jnp.int32)
        plsc.store_scatter(oi_v, [idx], ci, mask=last)
        plsc.store_scatter(oj_v, [idx], cj, mask=last)
        return _

    lax.fori_loop(0, BPW, body, 0)

    pltpu.sync_copy(oi_v, out_i_hbm.at[pl.ds(base, BPW)])
    pltpu.sync_copy(oj_v, out_j_hbm.at[pl.ds(base, BPW)])


@jax.jit
def _bpr_sc(user_ids, item_ids_i, item_ids_j, user_factors, item_factors):
    mesh = plsc.VectorSubcoreMesh(core_axis_name="c", subcore_axis_name="s")
    sweep = pl.kernel(
        _sweep_body,
        out_type=(jax.ShapeDtypeStruct((STAGE,), jnp.float32),
                  jax.ShapeDtypeStruct((STAGE,), jnp.float32),
                  jax.ShapeDtypeStruct((STAGE,), jnp.float32)),
        mesh=mesh,
        compiler_params=pltpu.CompilerParams(needs_layout_passes=False,
                                             use_tc_tiling_on_sc=True),
        scratch_types=[
            pltpu.VMEM((2048,), jnp.int32),
            pltpu.VMEM((HCAP,), jnp.int32), pltpu.VMEM((HCAP,), jnp.int32),
            pltpu.VMEM((HCAP,), jnp.int32), pltpu.VMEM((HCAP,), jnp.int32),
            pltpu.VMEM((HCAP,), jnp.int32), pltpu.VMEM((HCAP,), jnp.int32),
            pltpu.VMEM((TCAP,), jnp.int32), pltpu.VMEM((TCAP,), jnp.int32),
            pltpu.VMEM((TCAP,), jnp.int32), pltpu.VMEM((TCAP,), jnp.int32),
            pltpu.VMEM((TCAP,), jnp.int32), pltpu.VMEM((TCAP,), jnp.int32),
            pltpu.VMEM((HCAP,), jnp.int32), pltpu.VMEM((HCAP,), jnp.int32),
            pltpu.VMEM((D, CH), jnp.float32),
            pltpu.VMEM((128 * D,), jnp.float32),
            pltpu.VMEM((32, 128), jnp.int32),
            pltpu.SemaphoreType.DMA,
            pltpu.SemaphoreType.DMA,
        ],
    )
    uftT = user_factors.T[:, TAIL_LO:]
    ittT = item_factors.T[:, TAIL_LO:]
    us, is_, js = sweep(user_ids, item_ids_i, item_ids_j,
                        user_factors.T, item_factors.T, uftT, ittT)

    comp = pl.kernel(
        _compute_body,
        out_type=(jax.ShapeDtypeStruct((BATCH,), jnp.float32),
                  jax.ShapeDtypeStruct((BATCH,), jnp.float32)),
        mesh=mesh,
        compiler_params=pltpu.CompilerParams(needs_layout_passes=False,
                                             use_tc_tiling_on_sc=False),
        scratch_types=[
            pltpu.VMEM((BPW * D,), jnp.float32),
            pltpu.VMEM((BPW * D,), jnp.float32),
            pltpu.VMEM((BPW * D,), jnp.float32),
            pltpu.VMEM((BPW,), jnp.float32),
            pltpu.VMEM((BPW,), jnp.float32),
        ],
    )
    return comp(us, is_, js)


def kernel(user_ids, item_ids_i, item_ids_j, user_factors, item_factors):
    return _bpr_sc(user_ids, item_ids_i, item_ids_j,
                   user_factors, item_factors)


# speedup vs baseline: 7.8888x; 7.8863x over previous
import functools

import jax
import jax.numpy as jnp
from jax import lax
from jax.experimental import pallas as pl
from jax.experimental.pallas import tpu as pltpu
from jax.experimental.pallas import tpu_sc as plsc

D = 32
BATCH = 16384
NROWS = 1000000
NC = 2
NS = 16
NW = NC * NS
BPW = BATCH // NW
CH = 1024
NFULL = NROWS // CH
LAST_LO = NFULL * CH
TAIL0 = 999936
TAIL_LO = 999872
HCAP = 2048
TCAP = 256
STAGE = BATCH * D + NW * 128
TRASH = BATCH * D
TAIL_W = 17


def _sweep_body(uid_hbm, iid_hbm, jid_hbm, ufT_hbm, itfT_hbm,
                uftT_hbm, ittT_hbm,
                us_hbm, is_hbm, js_hbm,
                idb_v, hid_u, hb_u, hid_i, hb_i, hid_j, hb_j,
                tid_u, tb_u, tid_i, tb_i, tid_j, tb_j,
                chid_v, chb_v, slab_v, rows_v, ridx_v,
                sem, sem2):
    wid = lax.axis_index("s") * NC + lax.axis_index("c")
    lane = lax.iota(jnp.int32, 16)

    def prescan(list_hbm, hid, hb, tidl, tbl):
        def outer(ci, offs):
            pltpu.sync_copy(list_hbm.at[pl.ds(ci * 2048, 2048)], idb_v)

            def inner(g, offs):
                off, toff = offs
                ids = idb_v[pl.ds(g * 16, 16)]
                b = ci * 2048 + g * 16 + lane
                own = lax.shift_right_logical(ids, 10)
                m = (lax.bitwise_and(own, 31) == wid) & (ids < TAIL0)
                cnt = jnp.sum(m.astype(jnp.int32))

                @pl.when(cnt > 0)
                def _():
                    plsc.store_compressed(hid.at[pl.ds(off, 16)], ids, mask=m)
                    plsc.store_compressed(hb.at[pl.ds(off, 16)], b, mask=m)

                mt = (ids >= TAIL0) & (wid == TAIL_W)
                tcnt = jnp.sum(mt.astype(jnp.int32))

                @pl.when(tcnt > 0)
                def _():
                    plsc.store_compressed(tidl.at[pl.ds(toff, 16)], ids,
                                          mask=mt)
                    plsc.store_compressed(tbl.at[pl.ds(toff, 16)], b, mask=mt)

                return (jnp.minimum(off + cnt, HCAP - 16),
                        jnp.minimum(toff + tcnt, TCAP - 16))

            return lax.fori_loop(0, 128, inner, offs)

        return lax.fori_loop(0, 8, outer, (0, 0))

    n_u, nt_u = prescan(uid_hbm, hid_u, hb_u, tid_u, tb_u)
    n_i, nt_i = prescan(iid_hbm, hid_i, hb_i, tid_i, tb_i)
    n_j, nt_j = prescan(jid_hbm, hid_j, hb_j, tid_j, tb_j)

    trash0 = TRASH + wid * 128

    def reset_ridx():
        def rr(q, _):
            for h in range(8):
                ridx_v[q, pl.ds(h * 16, 16)] = trash0 + lane + h * 16
            return _
        lax.fori_loop(0, 32, rr, 0)

    def flush(stage_hbm, nrows):
        del nrows
        for q in range(32):
            pltpu.make_async_copy(
                rows_v.at[pl.ds(q * 128, 128)],
                stage_hbm.at[plsc.Indices(ridx_v.at[q])],
                sem2).start()
        pltpu.make_async_copy(
            rows_v, stage_hbm.at[pl.ds(0, 128 * D)], sem2).wait()

    def extract(stage_hbm, nchunk, lo):
        def wave(w, _):
            reset_ridx()
            base = w * 128
            nrows = jnp.minimum(nchunk - base, 128)

            def grp(g, _):
                valid = (g * 16 + lane) < nrows
                ids = chid_v[pl.ds(base + g * 16, 16)]
                bs = chb_v[pl.ds(base + g * 16, 16)]
                col = jnp.where(valid, ids - lo, 0)
                slot = g * 16 + lane
                for c in range(D):
                    cc = jnp.full((16,), c, jnp.int32)
                    vals = plsc.load_gather(slab_v, [cc, col], mask=valid)
                    fpos = slot * D + c
                    plsc.store_scatter(rows_v, [fpos], vals, mask=valid)
                    hpos = bs * D + c
                    plsc.store_scatter(
                        ridx_v,
                        [lax.shift_right_logical(fpos, 7),
                         lax.bitwise_and(fpos, 127)],
                        hpos, mask=valid)
                return _

            lax.fori_loop(0, lax.div(nrows + 15, 16), grp, 0)
            flush(stage_hbm, nrows)
            return _

        lax.fori_loop(0, lax.div(nchunk + 127, 128), wave, 0)

    def scanlist(hid, hb, n, t):
        def blk(bk, coff):
            valid = (bk * 16 + lane) < n
            ids = hid[pl.ds(bk * 16, 16)]
            bs = hb[pl.ds(bk * 16, 16)]
            m = valid & (lax.shift_right_logical(ids, 10) == t)
            cnt = jnp.sum(m.astype(jnp.int32))

            @pl.when(cnt > 0)
            def _():
                plsc.store_compressed(chid_v.at[pl.ds(coff, 16)], ids, mask=m)
                plsc.store_compressed(chb_v.at[pl.ds(coff, 16)], bs, mask=m)

            return coff + cnt

        return lax.fori_loop(0, lax.div(n + 15, 16), blk, 0)

    def copy_slab(table_hbm, lo, width):
        cps = []
        for c8 in range(D // 8):
            cps.append(pltpu.make_async_copy(
                table_hbm.at[pl.ds(c8 * 8, 8), pl.ds(lo, width)],
                slab_v.at[pl.ds(c8 * 8, 8), pl.ds(0, width)], sem))
        for cp in cps:
            cp.start()
        for cp in cps:
            cp.wait()

    def chunk_loop(s, carry):
        t = wid + 32 * s

        @pl.when(t < NFULL)
        def _():
            lo = pl.multiple_of(t * CH, CH)
            copy_slab(ufT_hbm, lo, CH)
            extract(us_hbm, scanlist(hid_u, hb_u, n_u, t), lo)
            copy_slab(itfT_hbm, lo, CH)
            extract(is_hbm, scanlist(hid_i, hb_i, n_i, t), lo)
            extract(js_hbm, scanlist(hid_j, hb_j, n_j, t), lo)

        return carry

    lax.fori_loop(0, (NFULL + NW - 1) // NW, chunk_loop, 0)

    @pl.when(wid == NFULL % NW)
    def _():
        copy_slab(ufT_hbm, LAST_LO, TAIL0 - LAST_LO)
        extract(us_hbm, scanlist(hid_u, hb_u, n_u, NFULL), LAST_LO)
        copy_slab(itfT_hbm, LAST_LO, TAIL0 - LAST_LO)
        extract(is_hbm, scanlist(hid_i, hb_i, n_i, NFULL), LAST_LO)
        extract(js_hbm, scanlist(hid_j, hb_j, n_j, NFULL), LAST_LO)

    @pl.when(wid == TAIL_W)
    def _():
        def dotail(tidl, tbl, nt, tslab_hbm, stage_hbm):
            pltpu.sync_copy(tslab_hbm, slab_v.at[:, pl.ds(0, 128)])

            def mk(g, _):
                chid_v[pl.ds(g * 16, 16)] = tidl[pl.ds(g * 16, 16)]
                chb_v[pl.ds(g * 16, 16)] = tbl[pl.ds(g * 16, 16)]
                return _

            lax.fori_loop(0, TCAP // 16, mk, 0)
            extract(stage_hbm, nt, TAIL_LO)

        dotail(tid_u, tb_u, nt_u, uftT_hbm, us_hbm)
        dotail(tid_i, tb_i, nt_i, ittT_hbm, is_hbm)
        dotail(tid_j, tb_j, nt_j, ittT_hbm, js_hbm)


def _compute_body(us_hbm, is_hbm, js_hbm, out_i_hbm, out_j_hbm,
                  su_v, si_v, sj_v, oi_v, oj_v):
    wid = lax.axis_index("s") * NC + lax.axis_index("c")
    base = wid * BPW
    pltpu.sync_copy(us_hbm.at[pl.ds(base * D, BPW * D)], su_v)
    pltpu.sync_copy(is_hbm.at[pl.ds(base * D, BPW * D)], si_v)
    pltpu.sync_copy(js_hbm.at[pl.ds(base * D, BPW * D)], sj_v)

    lane = lax.iota(jnp.int32, 16)
    last = lane == 15

    def body(r, _):
        u0 = su_v[pl.ds(r * D, 16)]
        u1 = su_v[pl.ds(r * D + 16, 16)]
        i0 = si_v[pl.ds(r * D, 16)]
        i1 = si_v[pl.ds(r * D + 16, 16)]
        j0 = sj_v[pl.ds(r * D, 16)]
        j1 = sj_v[pl.ds(r * D + 16, 16)]
        ci = plsc.cumsum(u0 * i0 + u1 * i1)
        cj = plsc.cumsum(u0 * j0 + u1 * j1)
        idx = jnp.full((16,), r, jnp.int32)
        plsc.store_scatter(oi_v, [idx], ci, mask=last)
        plsc.store_scatter(oj_v, [idx], cj, mask=last)
        return _

    lax.fori_loop(0, BPW, body, 0)

    pltpu.sync_copy(oi_v, out_i_hbm.at[pl.ds(base, BPW)])
    pltpu.sync_copy(oj_v, out_j_hbm.at[pl.ds(base, BPW)])


@jax.jit
def _bpr_sc(user_ids, item_ids_i, item_ids_j, user_factors, item_factors):
    mesh = plsc.VectorSubcoreMesh(core_axis_name="c", subcore_axis_name="s")
    sweep = pl.kernel(
        _sweep_body,
        out_type=(jax.ShapeDtypeStruct((STAGE,), jnp.float32),
                  jax.ShapeDtypeStruct((STAGE,), jnp.float32),
                  jax.ShapeDtypeStruct((STAGE,), jnp.float32)),
        mesh=mesh,
        compiler_params=pltpu.CompilerParams(needs_layout_passes=False,
                                             use_tc_tiling_on_sc=True),
        scratch_types=[
            pltpu.VMEM((2048,), jnp.int32),
            pltpu.VMEM((HCAP,), jnp.int32), pltpu.VMEM((HCAP,), jnp.int32),
            pltpu.VMEM((HCAP,), jnp.int32), pltpu.VMEM((HCAP,), jnp.int32),
            pltpu.VMEM((HCAP,), jnp.int32), pltpu.VMEM((HCAP,), jnp.int32),
            pltpu.VMEM((TCAP,), jnp.int32), pltpu.VMEM((TCAP,), jnp.int32),
            pltpu.VMEM((TCAP,), jnp.int32), pltpu.VMEM((TCAP,), jnp.int32),
            pltpu.VMEM((TCAP,), jnp.int32), pltpu.VMEM((TCAP,), jnp.int32),
            pltpu.VMEM((HCAP,), jnp.int32), pltpu.VMEM((HCAP,), jnp.int32),
            pltpu.VMEM((D, CH), jnp.float32),
            pltpu.VMEM((128 * D,), jnp.float32),
            pltpu.VMEM((32, 128), jnp.int32),
            pltpu.SemaphoreType.DMA,
            pltpu.SemaphoreType.DMA,
        ],
    )
    uftT = user_factors.T[:, TAIL_LO:]
    ittT = item_factors.T[:, TAIL_LO:]
    us, is_, js = sweep(user_ids, item_ids_i, item_ids_j,
                        user_factors.T, item_factors.T, uftT, ittT)

    comp = pl.kernel(
        _compute_body,
        out_type=(jax.ShapeDtypeStruct((BATCH,), jnp.float32),
                  jax.ShapeDtypeStruct((BATCH,), jnp.float32)),
        mesh=mesh,
        compiler_params=pltpu.CompilerParams(needs_layout_passes=False,
                                             use_tc_tiling_on_sc=False),
        scratch_types=[
            pltpu.VMEM((BPW * D,), jnp.float32),
            pltpu.VMEM((BPW * D,), jnp.float32),
            pltpu.VMEM((BPW * D,), jnp.float32),
            pltpu.VMEM((BPW,), jnp.float32),
            pltpu.VMEM((BPW,), jnp.float32),
        ],
    )
    return comp(us, is_, js)


def kernel(user_ids, item_ids_i, item_ids_j, user_factors, item_factors):
    return _bpr_sc(user_ids, item_ids_i, item_ids_j,
                   user_factors, item_factors)


# speedup vs baseline: 708.4806x; 89.8079x over previous
import functools

import jax
import jax.numpy as jnp
from jax import lax
from jax.experimental import pallas as pl
from jax.experimental.pallas import tpu as pltpu
from jax.experimental.pallas import tpu_sc as plsc

D = 32
BATCH = 16384
NROWS = 1000000
NC = 2
NS = 16
NW = NC * NS
BPW = BATCH // NW
CH = 1024
NFULL = NROWS // CH
LAST_LO = NFULL * CH
TAIL0 = 999936
TAIL_LO = 999872
HCAP = 2048
TCAP = 256
SROWS = BATCH + NW
SD = 128
TAIL_W = 17


def _sweep_body(uid_hbm, iid_hbm, jid_hbm, ufT_hbm, itfT_hbm,
                uftT_hbm, ittT_hbm,
                us_hbm, is_hbm, js_hbm,
                idb_v, hid_u, hb_u, hid_i, hb_i, hid_j, hb_j,
                tid_u, tb_u, tid_i, tb_i, tid_j, tb_j,
                chid_v, chb_v, slab_v, rows_v, ridx_v,
                sem, sem2):
    wid = lax.axis_index("s") * NC + lax.axis_index("c")
    lane = lax.iota(jnp.int32, 16)

    def prescan(list_hbm, hid, hb, tidl, tbl):
        def outer(ci, offs):
            pltpu.sync_copy(list_hbm.at[pl.ds(ci * 2048, 2048)], idb_v)

            def inner(g, offs):
                off, toff = offs
                ids = idb_v[pl.ds(g * 16, 16)]
                b = ci * 2048 + g * 16 + lane
                own = lax.shift_right_logical(ids, 10)
                m = (lax.bitwise_and(own, 31) == wid) & (ids < TAIL0)
                cnt = jnp.sum(m.astype(jnp.int32))

                @pl.when(cnt > 0)
                def _():
                    plsc.store_compressed(hid.at[pl.ds(off, 16)], ids, mask=m)
                    plsc.store_compressed(hb.at[pl.ds(off, 16)], b, mask=m)

                mt = (ids >= TAIL0) & (wid == TAIL_W)
                tcnt = jnp.sum(mt.astype(jnp.int32))

                @pl.when(tcnt > 0)
                def _():
                    plsc.store_compressed(tidl.at[pl.ds(toff, 16)], ids,
                                          mask=mt)
                    plsc.store_compressed(tbl.at[pl.ds(toff, 16)], b, mask=mt)

                return (jnp.minimum(off + cnt, HCAP - 16),
                        jnp.minimum(toff + tcnt, TCAP - 16))

            return lax.fori_loop(0, 128, inner, offs)

        return lax.fori_loop(0, 8, outer, (0, 0))

    n_u, nt_u = prescan(uid_hbm, hid_u, hb_u, tid_u, tb_u)
    n_i, nt_i = prescan(iid_hbm, hid_i, hb_i, tid_i, tb_i)
    n_j, nt_j = prescan(jid_hbm, hid_j, hb_j, tid_j, tb_j)

    trash_row = BATCH + wid

    def extract(stage_hbm, nchunk, lo):
        def wave(w, _):
            base = w * 128
            nrows = jnp.minimum(nchunk - base, 128)

            def rst(h, _):
                ridx_v[pl.ds(h * 16, 16)] = jnp.full((16,), trash_row,
                                                     jnp.int32)
                return _

            lax.fori_loop(0, 8, rst, 0)

            def grp(g, _):
                valid = (g * 16 + lane) < nrows
                ids = chid_v[pl.ds(base + g * 16, 16)]
                bs = chb_v[pl.ds(base + g * 16, 16)]
                col = jnp.where(valid, ids - lo, 0)
                slot = g * 16 + lane
                plsc.store_scatter(ridx_v, [slot], bs, mask=valid)
                for c in range(D):
                    cc = jnp.full((16,), c, jnp.int32)
                    vals = plsc.load_gather(slab_v, [cc, col], mask=valid)
                    plsc.store_scatter(rows_v, [slot, cc], vals, mask=valid)
                return _

            lax.fori_loop(0, lax.div(nrows + 15, 16), grp, 0)
            pltpu.sync_copy(rows_v, stage_hbm.at[plsc.Indices(ridx_v)])
            return _

        lax.fori_loop(0, lax.div(nchunk + 127, 128), wave, 0)

    def scanlist(hid, hb, n, t):
        def blk(bk, coff):
            valid = (bk * 16 + lane) < n
            ids = hid[pl.ds(bk * 16, 16)]
            bs = hb[pl.ds(bk * 16, 16)]
            m = valid & (lax.shift_right_logical(ids, 10) == t)
            cnt = jnp.sum(m.astype(jnp.int32))

            @pl.when(cnt > 0)
            def _():
                plsc.store_compressed(chid_v.at[pl.ds(coff, 16)], ids, mask=m)
                plsc.store_compressed(chb_v.at[pl.ds(coff, 16)], bs, mask=m)

            return coff + cnt

        return lax.fori_loop(0, lax.div(n + 15, 16), blk, 0)

    def copy_slab(table_hbm, lo, width):
        cps = []
        for c8 in range(D // 8):
            cps.append(pltpu.make_async_copy(
                table_hbm.at[pl.ds(c8 * 8, 8), pl.ds(lo, width)],
                slab_v.at[pl.ds(c8 * 8, 8), pl.ds(0, width)], sem))
        for cp in cps:
            cp.start()
        for cp in cps:
            cp.wait()

    def chunk_loop(s, carry):
        t = wid + 32 * s

        @pl.when(t < NFULL)
        def _():
            lo = pl.multiple_of(t * CH, CH)
            copy_slab(ufT_hbm, lo, CH)
            extract(us_hbm, scanlist(hid_u, hb_u, n_u, t), lo)
            copy_slab(itfT_hbm, lo, CH)
            extract(is_hbm, scanlist(hid_i, hb_i, n_i, t), lo)
            extract(js_hbm, scanlist(hid_j, hb_j, n_j, t), lo)

        return carry

    lax.fori_loop(0, (NFULL + NW - 1) // NW, chunk_loop, 0)

    @pl.when(wid == NFULL % NW)
    def _():
        copy_slab(ufT_hbm, LAST_LO, TAIL0 - LAST_LO)
        extract(us_hbm, scanlist(hid_u, hb_u, n_u, NFULL), LAST_LO)
        copy_slab(itfT_hbm, LAST_LO, TAIL0 - LAST_LO)
        extract(is_hbm, scanlist(hid_i, hb_i, n_i, NFULL), LAST_LO)
        extract(js_hbm, scanlist(hid_j, hb_j, n_j, NFULL), LAST_LO)

    @pl.when(wid == TAIL_W)
    def _():
        def dotail(tidl, tbl, nt, tslab_hbm, stage_hbm):
            pltpu.sync_copy(tslab_hbm, slab_v.at[:, pl.ds(0, 128)])

            def mk(g, _):
                chid_v[pl.ds(g * 16, 16)] = tidl[pl.ds(g * 16, 16)]
                chb_v[pl.ds(g * 16, 16)] = tbl[pl.ds(g * 16, 16)]
                return _

            lax.fori_loop(0, TCAP // 16, mk, 0)
            extract(stage_hbm, nt, TAIL_LO)

        dotail(tid_u, tb_u, nt_u, uftT_hbm, us_hbm)
        dotail(tid_i, tb_i, nt_i, ittT_hbm, is_hbm)
        dotail(tid_j, tb_j, nt_j, ittT_hbm, js_hbm)


def _compute_body(us_hbm, is_hbm, js_hbm, out_i_hbm, out_j_hbm,
                  su_v, si_v, sj_v, oi_v, oj_v):
    wid = lax.axis_index("s") * NC + lax.axis_index("c")
    base = wid * BPW
    pltpu.sync_copy(us_hbm.at[pl.ds(base, BPW), pl.ds(0, D)], su_v)
    pltpu.sync_copy(is_hbm.at[pl.ds(base, BPW), pl.ds(0, D)], si_v)
    pltpu.sync_copy(js_hbm.at[pl.ds(base, BPW), pl.ds(0, D)], sj_v)

    lane = lax.iota(jnp.int32, 16)
    last = lane == 15

    def body(r, _):
        u0 = su_v[r, pl.ds(0, 16)]
        u1 = su_v[r, pl.ds(16, 16)]
        i0 = si_v[r, pl.ds(0, 16)]
        i1 = si_v[r, pl.ds(16, 16)]
        j0 = sj_v[r, pl.ds(0, 16)]
        j1 = sj_v[r, pl.ds(16, 16)]
        ci = plsc.cumsum(u0 * i0 + u1 * i1)
        cj = plsc.cumsum(u0 * j0 + u1 * j1)
        idx = jnp.full((16,), r, jnp.int32)
        plsc.store_scatter(oi_v, [idx], ci, mask=last)
        plsc.store_scatter(oj_v, [idx], cj, mask=last)
        return _

    lax.fori_loop(0, BPW, body, 0)

    pltpu.sync_copy(oi_v, out_i_hbm.at[pl.ds(base, BPW)])
    pltpu.sync_copy(oj_v, out_j_hbm.at[pl.ds(base, BPW)])


@jax.jit
def _bpr_sc(user_ids, item_ids_i, item_ids_j, user_factors, item_factors):
    mesh = plsc.VectorSubcoreMesh(core_axis_name="c", subcore_axis_name="s")
    sweep = pl.kernel(
        _sweep_body,
        out_type=(jax.ShapeDtypeStruct((SROWS, SD), jnp.float32),
                  jax.ShapeDtypeStruct((SROWS, SD), jnp.float32),
                  jax.ShapeDtypeStruct((SROWS, SD), jnp.float32)),
        mesh=mesh,
        compiler_params=pltpu.CompilerParams(needs_layout_passes=False,
                                             use_tc_tiling_on_sc=True),
        scratch_types=[
            pltpu.VMEM((2048,), jnp.int32),
            pltpu.VMEM((HCAP,), jnp.int32), pltpu.VMEM((HCAP,), jnp.int32),
            pltpu.VMEM((HCAP,), jnp.int32), pltpu.VMEM((HCAP,), jnp.int32),
            pltpu.VMEM((HCAP,), jnp.int32), pltpu.VMEM((HCAP,), jnp.int32),
            pltpu.VMEM((TCAP,), jnp.int32), pltpu.VMEM((TCAP,), jnp.int32),
            pltpu.VMEM((TCAP,), jnp.int32), pltpu.VMEM((TCAP,), jnp.int32),
            pltpu.VMEM((TCAP,), jnp.int32), pltpu.VMEM((TCAP,), jnp.int32),
            pltpu.VMEM((HCAP,), jnp.int32), pltpu.VMEM((HCAP,), jnp.int32),
            pltpu.VMEM((D, CH), jnp.float32),
            pltpu.VMEM((128, SD), jnp.float32),
            pltpu.VMEM((128,), jnp.int32),
            pltpu.SemaphoreType.DMA,
            pltpu.SemaphoreType.DMA,
        ],
    )
    uftT = user_factors.T[:, TAIL_LO:]
    ittT = item_factors.T[:, TAIL_LO:]
    us, is_, js = sweep(user_ids, item_ids_i, item_ids_j,
                        user_factors.T, item_factors.T, uftT, ittT)

    comp = pl.kernel(
        _compute_body,
        out_type=(jax.ShapeDtypeStruct((BATCH,), jnp.float32),
                  jax.ShapeDtypeStruct((BATCH,), jnp.float32)),
        mesh=mesh,
        compiler_params=pltpu.CompilerParams(needs_layout_passes=False,
                                             use_tc_tiling_on_sc=False),
        scratch_types=[
            pltpu.VMEM((BPW, D), jnp.float32),
            pltpu.VMEM((BPW, D), jnp.float32),
            pltpu.VMEM((BPW, D), jnp.float32),
            pltpu.VMEM((BPW,), jnp.float32),
            pltpu.VMEM((BPW,), jnp.float32),
        ],
    )
    return comp(us, is_, js)


def kernel(user_ids, item_ids_i, item_ids_j, user_factors, item_factors):
    return _bpr_sc(user_ids, item_ids_i, item_ids_j,
                   user_factors, item_factors)


# speedup vs baseline: 713.8957x; 1.0076x over previous
import functools

import jax
import jax.numpy as jnp
from jax import lax
from jax.experimental import pallas as pl
from jax.experimental.pallas import tpu as pltpu
from jax.experimental.pallas import tpu_sc as plsc

D = 32
BATCH = 16384
NROWS = 1000000
NC = 2
NS = 16
NW = NC * NS
BPW = BATCH // NW
CH = 1024
NFULL = NROWS // CH
LAST_LO = NFULL * CH
TAIL0 = 999936
TAIL_LO = 999872
HCAP = 2048
TCAP = 256
SROWS = BATCH + NW
SD = 128
TAIL_W = 17


def _sweep_body(uid_hbm, iid_hbm, jid_hbm, ufT_hbm, itfT_hbm,
                uftT_hbm, ittT_hbm,
                us_hbm, is_hbm, js_hbm,
                idb_v, hid_u, hb_u, hid_i, hb_i, hid_j, hb_j,
                chid_v, chb_v, slab_v, rows_v, ridx_v,
                sem, sem2):
    wid = lax.axis_index("s") * NC + lax.axis_index("c")
    lane = lax.iota(jnp.int32, 16)

    def prescan(list_hbm, hid, hb):
        def outer(ci, off):
            pltpu.sync_copy(list_hbm.at[pl.ds(ci * 2048, 2048)], idb_v)

            def inner(g, off):
                ids = idb_v[pl.ds(g * 16, 16)]
                b = ci * 2048 + g * 16 + lane
                own = lax.bitwise_and(lax.shift_right_logical(ids, 10), 31)
                own = jnp.where(ids >= TAIL0, TAIL_W, own)
                m = own == wid
                cnt = jnp.sum(m.astype(jnp.int32))

                @pl.when(cnt > 0)
                def _():
                    plsc.store_compressed(hid.at[pl.ds(off, 16)], ids, mask=m)
                    plsc.store_compressed(hb.at[pl.ds(off, 16)], b, mask=m)

                return jnp.minimum(off + cnt, HCAP - 16)

            return lax.fori_loop(0, 128, inner, off)

        return lax.fori_loop(0, 8, outer, 0)

    n_u = prescan(uid_hbm, hid_u, hb_u)
    n_i = prescan(iid_hbm, hid_i, hb_i)
    n_j = prescan(jid_hbm, hid_j, hb_j)

    trash_row = BATCH + wid

    def extract(stage_hbm, nchunk, lo):
        def wave(w, _):
            base = w * 128
            nrows = jnp.minimum(nchunk - base, 128)

            def rst(h, _):
                ridx_v[pl.ds(h * 16, 16)] = jnp.full((16,), trash_row,
                                                     jnp.int32)
                return _

            lax.fori_loop(0, 8, rst, 0)

            def grp(g, _):
                valid = (g * 16 + lane) < nrows
                ids = chid_v[pl.ds(base + g * 16, 16)]
                bs = chb_v[pl.ds(base + g * 16, 16)]
                col = jnp.where(valid, ids - lo, 0)
                slot = g * 16 + lane
                plsc.store_scatter(ridx_v, [slot], bs, mask=valid)
                for c in range(D):
                    cc = jnp.full((16,), c, jnp.int32)
                    vals = plsc.load_gather(slab_v, [cc, col], mask=valid)
                    plsc.store_scatter(rows_v, [slot, cc], vals, mask=valid)
                return _

            lax.fori_loop(0, lax.div(nrows + 15, 16), grp, 0)
            pltpu.sync_copy(rows_v, stage_hbm.at[plsc.Indices(ridx_v)])
            return _

        lax.fori_loop(0, lax.div(nchunk + 127, 128), wave, 0)

    def scanlist(hid, hb, n, t):
        def blk(bk, coff):
            valid = (bk * 16 + lane) < n
            ids = hid[pl.ds(bk * 16, 16)]
            bs = hb[pl.ds(bk * 16, 16)]
            m = valid & (lax.shift_right_logical(ids, 10) == t)
            cnt = jnp.sum(m.astype(jnp.int32))

            @pl.when(cnt > 0)
            def _():
                plsc.store_compressed(chid_v.at[pl.ds(coff, 16)], ids, mask=m)
                plsc.store_compressed(chb_v.at[pl.ds(coff, 16)], bs, mask=m)

            return coff + cnt

        return lax.fori_loop(0, lax.div(n + 15, 16), blk, 0)

    def copy_slab(table_hbm, lo, width):
        cps = []
        for c8 in range(D // 8):
            cps.append(pltpu.make_async_copy(
                table_hbm.at[pl.ds(c8 * 8, 8), pl.ds(lo, width)],
                slab_v.at[pl.ds(c8 * 8, 8), pl.ds(0, width)], sem))
        for cp in cps:
            cp.start()
        for cp in cps:
            cp.wait()

    def chunk_loop(s, carry):
        t = wid + 32 * s

        @pl.when(t < NFULL)
        def _():
            lo = pl.multiple_of(t * CH, CH)
            copy_slab(ufT_hbm, lo, CH)
            extract(us_hbm, scanlist(hid_u, hb_u, n_u, t), lo)
            copy_slab(itfT_hbm, lo, CH)
            extract(is_hbm, scanlist(hid_i, hb_i, n_i, t), lo)
            extract(js_hbm, scanlist(hid_j, hb_j, n_j, t), lo)

        return carry

    lax.fori_loop(0, (NFULL + NW - 1) // NW, chunk_loop, 0)

    @pl.when(wid == NFULL % NW)
    def _():
        copy_slab(ufT_hbm, LAST_LO, TAIL0 - LAST_LO)
        extract(us_hbm, scanlist(hid_u, hb_u, n_u, NFULL), LAST_LO)
        copy_slab(itfT_hbm, LAST_LO, TAIL0 - LAST_LO)
        extract(is_hbm, scanlist(hid_i, hb_i, n_i, NFULL), LAST_LO)
        extract(js_hbm, scanlist(hid_j, hb_j, n_j, NFULL), LAST_LO)

    @pl.when(wid == TAIL_W)
    def _():
        def dotail(hid, hb, n, tslab_hbm, stage_hbm):
            pltpu.sync_copy(tslab_hbm, slab_v.at[:, pl.ds(0, 128)])
            extract(stage_hbm, scanlist(hid, hb, n, NFULL), TAIL_LO)

        dotail(hid_u, hb_u, n_u, uftT_hbm, us_hbm)
        dotail(hid_i, hb_i, n_i, ittT_hbm, is_hbm)
        dotail(hid_j, hb_j, n_j, ittT_hbm, js_hbm)


def _compute_body(us_hbm, is_hbm, js_hbm, out_i_hbm, out_j_hbm,
                  su_v, si_v, sj_v, oi_v, oj_v):
    wid = lax.axis_index("s") * NC + lax.axis_index("c")
    base = wid * BPW
    pltpu.sync_copy(us_hbm.at[pl.ds(base, BPW), pl.ds(0, D)], su_v)
    pltpu.sync_copy(is_hbm.at[pl.ds(base, BPW), pl.ds(0, D)], si_v)
    pltpu.sync_copy(js_hbm.at[pl.ds(base, BPW), pl.ds(0, D)], sj_v)

    lane = lax.iota(jnp.int32, 16)
    last = lane == 15

    def body(r, _):
        u0 = su_v[r, pl.ds(0, 16)]
        u1 = su_v[r, pl.ds(16, 16)]
        i0 = si_v[r, pl.ds(0, 16)]
        i1 = si_v[r, pl.ds(16, 16)]
        j0 = sj_v[r, pl.ds(0, 16)]
        j1 = sj_v[r, pl.ds(16, 16)]
        ci = plsc.cumsum(u0 * i0 + u1 * i1)
        cj = plsc.cumsum(u0 * j0 + u1 * j1)
        idx = jnp.full((16,), r, jnp.int32)
        plsc.store_scatter(oi_v, [idx], ci, mask=last)
        plsc.store_scatter(oj_v, [idx], cj, mask=last)
        return _

    lax.fori_loop(0, BPW, body, 0)

    pltpu.sync_copy(oi_v, out_i_hbm.at[pl.ds(base, BPW)])
    pltpu.sync_copy(oj_v, out_j_hbm.at[pl.ds(base, BPW)])


@jax.jit
def _bpr_sc(user_ids, item_ids_i, item_ids_j, user_factors, item_factors):
    mesh = plsc.VectorSubcoreMesh(core_axis_name="c", subcore_axis_name="s")
    sweep = pl.kernel(
        _sweep_body,
        out_type=(jax.ShapeDtypeStruct((SROWS, SD), jnp.float32),
                  jax.ShapeDtypeStruct((SROWS, SD), jnp.float32),
                  jax.ShapeDtypeStruct((SROWS, SD), jnp.float32)),
        mesh=mesh,
        compiler_params=pltpu.CompilerParams(needs_layout_passes=False,
                                             use_tc_tiling_on_sc=True),
        scratch_types=[
            pltpu.VMEM((2048,), jnp.int32),
            pltpu.VMEM((HCAP,), jnp.int32), pltpu.VMEM((HCAP,), jnp.int32),
            pltpu.VMEM((HCAP,), jnp.int32), pltpu.VMEM((HCAP,), jnp.int32),
            pltpu.VMEM((HCAP,), jnp.int32), pltpu.VMEM((HCAP,), jnp.int32),
            pltpu.VMEM((HCAP,), jnp.int32), pltpu.VMEM((HCAP,), jnp.int32),
            pltpu.VMEM((D, CH), jnp.float32),
            pltpu.VMEM((128, SD), jnp.float32),
            pltpu.VMEM((128,), jnp.int32),
            pltpu.SemaphoreType.DMA,
            pltpu.SemaphoreType.DMA,
        ],
    )
    uftT = user_factors.T[:, TAIL_LO:]
    ittT = item_factors.T[:, TAIL_LO:]
    us, is_, js = sweep(user_ids, item_ids_i, item_ids_j,
                        user_factors.T, item_factors.T, uftT, ittT)

    comp = pl.kernel(
        _compute_body,
        out_type=(jax.ShapeDtypeStruct((BATCH,), jnp.float32),
                  jax.ShapeDtypeStruct((BATCH,), jnp.float32)),
        mesh=mesh,
        compiler_params=pltpu.CompilerParams(needs_layout_passes=False,
                                             use_tc_tiling_on_sc=False),
        scratch_types=[
            pltpu.VMEM((BPW, D), jnp.float32),
            pltpu.VMEM((BPW, D), jnp.float32),
            pltpu.VMEM((BPW, D), jnp.float32),
            pltpu.VMEM((BPW,), jnp.float32),
            pltpu.VMEM((BPW,), jnp.float32),
        ],
    )
    return comp(us, is_, js)


def kernel(user_ids, item_ids_i, item_ids_j, user_factors, item_factors):
    return _bpr_sc(user_ids, item_ids_i, item_ids_j,
                   user_factors, item_factors)
